# baseline (speedup 1.0000x reference)
import functools

import jax
import jax.numpy as jnp
from jax import lax
from jax.experimental import pallas as pl
from jax.experimental.pallas import tpu as pltpu
from jax.experimental.pallas import tpu_sc as plsc

_NC = 2
_NS = 16
_NW = _NC * _NS
_SUB = 128
_NBUF = 8


def _sc_gather(idx2d, W):
    n_chunks, sub = idx2d.shape
    D = W.shape[1]
    B = n_chunks * sub
    chunks_w = n_chunks // _NW
    ngroups = chunks_w // _NBUF

    mesh = plsc.VectorSubcoreMesh(core_axis_name="c", subcore_axis_name="s")

    @functools.partial(
        pl.kernel,
        out_type=jax.ShapeDtypeStruct((B, D), jnp.float32),
        mesh=mesh,
        scratch_types=[
            pltpu.VMEM((_NBUF, _SUB), jnp.int32),
            pltpu.VMEM((_NBUF, _SUB, D), jnp.float32),
            pltpu.SemaphoreType.DMA,
            pltpu.SemaphoreType.DMA,
        ],
        compiler_params=pltpu.CompilerParams(use_tc_tiling_on_sc=False),
    )
    def k(idx_hbm, w_hbm, out_hbm, idx_v, rows_v, gsem, ssem):
        wid = lax.axis_index("s") * _NC + lax.axis_index("c")
        base_chunk = wid * chunks_w

        def group(g, carry):
            gbase = base_chunk + g * _NBUF
            pltpu.sync_copy(idx_hbm.at[pl.ds(gbase, _NBUF)], idx_v)
            gh = [
                pltpu.async_copy(w_hbm.at[idx_v.at[j]], rows_v.at[j], gsem)
                for j in range(_NBUF)
            ]
            sh = []
            for j in range(_NBUF):
                gh[j].wait()
                sh.append(
                    pltpu.async_copy(
                        rows_v.at[j],
                        out_hbm.at[pl.ds((gbase + j) * _SUB, _SUB)],
                        ssem,
                    )
                )
            for h in sh:
                h.wait()
            return carry

        lax.fori_loop(0, ngroups, group, 0)

    return k(idx2d, W)


def kernel(global_frame_idx, W):
    S, T = global_frame_idx.shape
    D = W.shape[1]
    B = S * T
    idx2d = global_frame_idx.astype(jnp.int32).reshape(B // _SUB, _SUB)
    out = _sc_gather(idx2d, W)
    return out.reshape(S, T, D)


# speedup vs baseline: 1.0120x; 1.0120x over previous
import functools

import jax
import jax.numpy as jnp
from jax import lax
from jax.experimental import pallas as pl
from jax.experimental.pallas import tpu as pltpu
from jax.experimental.pallas import tpu_sc as plsc

_NC = 2
_NS = 16
_NW = _NC * _NS
_SUB = 128
_K = 12
_G = 6
_MEGA = 40


def _sc_gather(idx2d, W):
    n_chunks, sub = idx2d.shape
    D = W.shape[1]
    B = n_chunks * sub
    nsub = n_chunks // _NW
    nblk = nsub // _MEGA

    mesh = plsc.VectorSubcoreMesh(core_axis_name="c", subcore_axis_name="s")

    @functools.partial(
        pl.kernel,
        out_type=jax.ShapeDtypeStruct((B, D), jnp.float32),
        mesh=mesh,
        scratch_types=[
            pltpu.VMEM((2, _MEGA, _SUB), jnp.int32),
            pltpu.VMEM((_K, _SUB, D), jnp.float32),
            pltpu.SemaphoreType.DMA,
            pltpu.SemaphoreType.DMA,
            pltpu.SemaphoreType.DMA,
        ],
        compiler_params=pltpu.CompilerParams(use_tc_tiling_on_sc=False),
    )
    def k(idx_hbm, w_hbm, out_hbm, idx_v, rows_v, isem, gsem, ssem):
        wid = lax.axis_index("s") * _NC + lax.axis_index("c")
        base_sub = wid * nsub

        def wait_idx():
            pltpu.make_async_copy(
                idx_hbm.at[pl.ds(base_sub, _MEGA)], idx_v.at[0], isem
            ).wait()

        def wait_gather():
            pltpu.make_async_copy(
                w_hbm.at[idx_v.at[0, 0]], rows_v.at[0], gsem
            ).wait()

        def wait_store():
            pltpu.make_async_copy(
                rows_v.at[0], out_hbm.at[pl.ds(0, _SUB)], ssem
            ).wait()

        def fire_store(j, slot):
            pltpu.async_copy(
                rows_v.at[slot],
                out_hbm.at[pl.ds((base_sub + j) * _SUB, _SUB)],
                ssem,
            )

        pltpu.async_copy(idx_hbm.at[pl.ds(base_sub, _MEGA)], idx_v.at[0], isem)

        def body(i, carry):
            s = i % _K
            blk = i // _MEGA
            q = blk % 2
            r = i % _MEGA

            @pl.when(r == 0)
            def _():
                wait_idx()

            @pl.when(jnp.logical_and(r == _G, blk + 1 < nblk))
            def _():
                pltpu.async_copy(
                    idx_hbm.at[pl.ds(base_sub + (blk + 1) * _MEGA, _MEGA)],
                    idx_v.at[1 - q],
                    isem,
                )

            @pl.when(i >= _K)
            def _():
                wait_store()

            pltpu.async_copy(w_hbm.at[idx_v.at[q, r]], rows_v.at[s], gsem)

            @pl.when(i >= _G)
            def _():
                wait_gather()
                fire_store(i - _G, (i - _G) % _K)

            return carry

        lax.fori_loop(0, nsub, body, 0)

        for t in range(_G):
            j = nsub - _G + t
            wait_gather()
            fire_store(j, j % _K)
        for _t in range(_K):
            wait_store()

    return k(idx2d, W)


def kernel(global_frame_idx, W):
    S, T = global_frame_idx.shape
    D = W.shape[1]
    B = S * T
    idx2d = global_frame_idx.astype(jnp.int32).reshape(B // _SUB, _SUB)
    out = _sc_gather(idx2d, W)
    return out.reshape(S, T, D)
